# baseline (speedup 1.0000x reference)
import functools

import jax
import jax.numpy as jnp
from jax import lax
from jax.experimental import pallas as pl
from jax.experimental.pallas import tpu as pltpu

_N = 20000
_PRE_MAX = 4096
_POST_MAX = 500
_SCORE_THRESH = 0.1
_NMS_THRESH = 0.5
_B = 256
_NB = _PRE_MAX // _B


def _pair_sup(x1r, x2r, y1r, y2r, ar, x1c, x2c, y1c, y2c, ac):
    iw = jnp.maximum(jnp.minimum(x2r, x2c) - jnp.maximum(x1r, x1c), 0.0)
    ih = jnp.maximum(jnp.minimum(y2r, y2c) - jnp.maximum(y1r, y1c), 0.0)
    inter = iw * ih
    union = ar + ac - inter
    iou = inter / jnp.maximum(union, 1e-6)
    return iou


def _nms_body(geo_r_ref, geo_c_ref, scores_ref, out_ref, keep_ref):
    scores = scores_ref[...]
    keep_ref[...] = (scores > _SCORE_THRESH).astype(jnp.float32)

    for b in range(_NB):
        r0 = b * _B
        w = _PRE_MAX - r0
        blk = geo_r_ref[r0:r0 + _B, :]
        x1r = blk[:, 0:1]
        x2r = blk[:, 1:2]
        y1r = blk[:, 2:3]
        y2r = blk[:, 3:4]
        ar = blk[:, 4:5]
        row_ids = r0 + lax.broadcasted_iota(jnp.int32, (_B, 1), 0)
        col_ids = r0 + lax.broadcasted_iota(jnp.int32, (1, w), 1)
        iou = _pair_sup(
            x1r, x2r, y1r, y2r, ar,
            geo_c_ref[0:1, r0:], geo_c_ref[1:2, r0:], geo_c_ref[2:3, r0:],
            geo_c_ref[3:4, r0:], geo_c_ref[4:5, r0:])
        sup = ((iou > _NMS_THRESH) & (row_ids < col_ids)).astype(jnp.float32)
        sup_intra = sup[:, 0:_B]

        kb0 = keep_ref[0:1, r0:r0 + _B]

        def fp_cond(c):
            return c[1]

        def fp_body(c, kb0=kb0, sup_intra=sup_intra):
            kb, _ = c
            hit = lax.dot_general(
                kb, sup_intra, (((1,), (0,)), ((), ())),
                preferred_element_type=jnp.float32)
            kb_new = kb0 * (hit < 0.5).astype(jnp.float32)
            return kb_new, jnp.any(kb_new != kb)

        kb, _ = lax.while_loop(fp_cond, fp_body, (kb0, True))

        hit_all = lax.dot_general(
            kb, sup, (((1,), (0,)), ((), ())),
            preferred_element_type=jnp.float32)
        keep_ref[0:1, r0:] = keep_ref[0:1, r0:] * (
            hit_all < 0.5).astype(jnp.float32)

    out_ref[...] = jnp.where(keep_ref[...] > 0.0, scores, -1.0)


@functools.partial(jax.jit)
def kernel(box_preds, cls_preds):
    rank_scores = jnp.max(cls_preds, axis=-1)
    normalized_scores = jax.nn.sigmoid(rank_scores)
    top_scores, top_idx = lax.top_k(normalized_scores, _PRE_MAX)
    top_boxes = box_preds[top_idx]

    b = lax.stop_gradient(top_boxes)
    c = jnp.abs(jnp.cos(b[:, 6]))
    s = jnp.abs(jnp.sin(b[:, 6]))
    hx = 0.5 * (jnp.abs(b[:, 3]) * c + jnp.abs(b[:, 4]) * s)
    hy = 0.5 * (jnp.abs(b[:, 3]) * s + jnp.abs(b[:, 4]) * c)
    x1 = b[:, 0] - hx
    x2 = b[:, 0] + hx
    y1 = b[:, 1] - hy
    y2 = b[:, 1] + hy
    area = (x2 - x1) * (y2 - y1)
    zeros = jnp.zeros_like(x1)
    geo_c = jnp.stack([x1, x2, y1, y2, area, zeros, zeros, zeros], axis=0)
    geo_r = geo_c.T

    sel_scores = pl.pallas_call(
        _nms_body,
        out_shape=jax.ShapeDtypeStruct((1, _PRE_MAX), jnp.float32),
        scratch_shapes=[pltpu.VMEM((1, _PRE_MAX), jnp.float32)],
    )(geo_r, geo_c, top_scores[None, :])[0]

    final_scores, sel = lax.top_k(sel_scores, _POST_MAX)
    final_boxes = top_boxes[sel]
    return jnp.concatenate([final_boxes, final_scores[:, None]], axis=-1)


# speedup vs baseline: 1.0484x; 1.0484x over previous
import functools

import jax
import jax.numpy as jnp
from jax import lax
from jax.experimental import pallas as pl
from jax.experimental.pallas import tpu as pltpu

_N = 20000
_PRE_MAX = 4096
_POST_MAX = 500
_SCORE_THRESH = 0.1
_NMS_THRESH = 0.5
_B = 1024
_NB = _PRE_MAX // _B


def _pair_sup(x1r, x2r, y1r, y2r, ar, x1c, x2c, y1c, y2c, ac):
    iw = jnp.maximum(jnp.minimum(x2r, x2c) - jnp.maximum(x1r, x1c), 0.0)
    ih = jnp.maximum(jnp.minimum(y2r, y2c) - jnp.maximum(y1r, y1c), 0.0)
    inter = iw * ih
    union = ar + ac - inter
    iou = inter / jnp.maximum(union, 1e-6)
    return iou


def _nms_body(geo_r_ref, geo_c_ref, scores_ref, out_ref, keep_ref):
    scores = scores_ref[...]
    keep_ref[...] = (scores > _SCORE_THRESH).astype(jnp.float32)

    for b in range(_NB):
        r0 = b * _B
        w = _PRE_MAX - r0
        blk = geo_r_ref[r0:r0 + _B, :]
        x1r = blk[:, 0:1]
        x2r = blk[:, 1:2]
        y1r = blk[:, 2:3]
        y2r = blk[:, 3:4]
        ar = blk[:, 4:5]
        row_ids = r0 + lax.broadcasted_iota(jnp.int32, (_B, 1), 0)
        col_ids = r0 + lax.broadcasted_iota(jnp.int32, (1, w), 1)
        iou = _pair_sup(
            x1r, x2r, y1r, y2r, ar,
            geo_c_ref[0:1, r0:], geo_c_ref[1:2, r0:], geo_c_ref[2:3, r0:],
            geo_c_ref[3:4, r0:], geo_c_ref[4:5, r0:])
        sup = ((iou > _NMS_THRESH) & (row_ids < col_ids)).astype(jnp.float32)
        sup_intra = sup[:, 0:_B]

        kb0 = keep_ref[0:1, r0:r0 + _B]

        def fp_cond(c):
            return c[1]

        def fp_body(c, kb0=kb0, sup_intra=sup_intra):
            kb, _ = c
            hit = lax.dot_general(
                kb, sup_intra, (((1,), (0,)), ((), ())),
                preferred_element_type=jnp.float32)
            kb_new = kb0 * (hit < 0.5).astype(jnp.float32)
            return kb_new, jnp.any(kb_new != kb)

        kb, _ = lax.while_loop(fp_cond, fp_body, (kb0, True))

        hit_all = lax.dot_general(
            kb, sup, (((1,), (0,)), ((), ())),
            preferred_element_type=jnp.float32)
        keep_ref[0:1, r0:] = keep_ref[0:1, r0:] * (
            hit_all < 0.5).astype(jnp.float32)

    out_ref[...] = jnp.where(keep_ref[...] > 0.0, scores, -1.0)


@functools.partial(jax.jit)
def kernel(box_preds, cls_preds):
    rank_scores = jnp.max(cls_preds, axis=-1)
    normalized_scores = jax.nn.sigmoid(rank_scores)
    top_scores, top_idx = lax.top_k(normalized_scores, _PRE_MAX)
    top_boxes = box_preds[top_idx]

    b = lax.stop_gradient(top_boxes)
    c = jnp.abs(jnp.cos(b[:, 6]))
    s = jnp.abs(jnp.sin(b[:, 6]))
    hx = 0.5 * (jnp.abs(b[:, 3]) * c + jnp.abs(b[:, 4]) * s)
    hy = 0.5 * (jnp.abs(b[:, 3]) * s + jnp.abs(b[:, 4]) * c)
    x1 = b[:, 0] - hx
    x2 = b[:, 0] + hx
    y1 = b[:, 1] - hy
    y2 = b[:, 1] + hy
    area = (x2 - x1) * (y2 - y1)
    zeros = jnp.zeros_like(x1)
    geo_c = jnp.stack([x1, x2, y1, y2, area, zeros, zeros, zeros], axis=0)
    geo_r = geo_c.T

    sel_scores = pl.pallas_call(
        _nms_body,
        out_shape=jax.ShapeDtypeStruct((1, _PRE_MAX), jnp.float32),
        scratch_shapes=[pltpu.VMEM((1, _PRE_MAX), jnp.float32)],
    )(geo_r, geo_c, top_scores[None, :])[0]

    final_scores, sel = lax.top_k(sel_scores, _POST_MAX)
    final_boxes = top_boxes[sel]
    return jnp.concatenate([final_boxes, final_scores[:, None]], axis=-1)
